# baseline (speedup 1.0000x reference)
import jax
from jax.experimental import pallas as pl
from jax.experimental.pallas import tpu as pltpu


def _copy_kernel(x_ref, o_ref, sem):
    copy = pltpu.make_async_copy(x_ref, o_ref, sem)
    copy.start()
    copy.wait()


def kernel(x, s):
    return pl.pallas_call(
        _copy_kernel,
        out_shape=jax.ShapeDtypeStruct(x.shape, x.dtype),
        in_specs=[pl.BlockSpec(memory_space=pl.ANY)],
        out_specs=pl.BlockSpec(memory_space=pl.ANY),
        scratch_shapes=[pltpu.SemaphoreType.DMA],
    )(x)


# speedup vs baseline: 48.9915x; 48.9915x over previous
import jax
from jax.experimental import pallas as pl


_ROWS = 1024


def _copy_kernel(x_ref, o_ref):
    o_ref[...] = x_ref[...]


def kernel(x, s):
    total_rows = x.shape[0] * x.shape[1]
    x2 = x.reshape(total_rows, x.shape[2])
    out = pl.pallas_call(
        _copy_kernel,
        grid=(total_rows // _ROWS,),
        in_specs=[pl.BlockSpec((_ROWS, x.shape[2]), lambda i: (i, 0))],
        out_specs=pl.BlockSpec((_ROWS, x.shape[2]), lambda i: (i, 0)),
        out_shape=jax.ShapeDtypeStruct(x2.shape, x.dtype),
    )(x2)
    return out.reshape(x.shape)
